# Initial kernel scaffold; baseline (speedup 1.0000x reference)
import functools

import jax
import jax.numpy as jnp
from jax import lax
from jax.experimental import pallas as pl
from jax.experimental.pallas import tpu as pltpu
from jax.experimental.pallas import tpu_sc as plsc

N = 10000
E = 320000
D = 128
NC, NS, L = 2, 16, 16
K = 80
E_CORE = E // NC
E_TILE = E_CORE // NS
E_TILE_DEG = E // NS
DEG_CHUNKS = E_TILE_DEG // K
MAIN_CHUNKS = E_TILE // K
RPT = N // NS


def _sc_body(src_hbm, dst_hbm, ew_hbm, h_hbm, part_hbm,
             deg_sh, acc_sh, z2d, z1d, sidx, didx, ewv, coef, rowbuf,
             degt, dinv):
    c = lax.axis_index("c")
    s = lax.axis_index("s")
    zero16 = jnp.zeros((L,), jnp.float32)

    @pl.loop(0, 125)
    def _(r):
        for k in range(D // L):
            z2d[r, pl.ds(k * L, L)] = zero16

    @pl.loop(0, 2000 // L)
    def _(i):
        z1d[pl.ds(i * L, L)] = zero16

    row0 = s * RPT
    for k in range(RPT // 125):
        pltpu.sync_copy(z2d, acc_sh.at[pl.ds(row0 + k * 125, 125)])

    @pl.when(s == 0)
    def _():
        for k in range(N // 2000):
            pltpu.sync_copy(z1d, deg_sh.at[pl.ds(k * 2000, 2000)])

    plsc.subcore_barrier()

    dbase = s * E_TILE_DEG

    @pl.loop(0, DEG_CHUNKS)
    def _(i):
        off = dbase + i * K
        pltpu.sync_copy(dst_hbm.at[pl.ds(off, K)], didx)
        pltpu.sync_copy(ew_hbm.at[pl.ds(off, K)], ewv)
        for g in range(K // L):
            sl = pl.ds(g * L, L)
            ewv[sl] = jnp.maximum(ewv[sl], 1e-5)
        pltpu.sync_copy(ewv, deg_sh.at[didx], add=True)

    plsc.subcore_barrier()

    pltpu.sync_copy(deg_sh, degt)

    @pl.loop(0, N // L)
    def _(g):
        sl = pl.ds(g * L, L)
        dv = degt[sl]
        x = jnp.maximum(dv, 1e-12)
        i32 = plsc.bitcast(x, jnp.int32)
        y = plsc.bitcast(jnp.int32(0x5F3759DF) - (i32 >> 1), jnp.float32)
        for _ in range(3):
            y = y * (1.5 - 0.5 * x * y * y)
        dinv[sl] = jnp.where(dv > 0.0, y, 0.0)

    mbase = c * E_CORE + s * E_TILE

    @pl.loop(0, MAIN_CHUNKS)
    def _(i):
        off = mbase + i * K
        pltpu.sync_copy(src_hbm.at[pl.ds(off, K)], sidx)
        pltpu.sync_copy(dst_hbm.at[pl.ds(off, K)], didx)
        pltpu.sync_copy(ew_hbm.at[pl.ds(off, K)], ewv)
        pltpu.sync_copy(h_hbm.at[sidx], rowbuf)
        for g in range(K // L):
            sl = pl.ds(g * L, L)
            e = jnp.maximum(ewv[sl], 1e-5)
            a = plsc.load_gather(dinv, [sidx[sl]])
            t = plsc.load_gather(dinv, [didx[sl]])
            coef[sl] = e * a * t

        @pl.loop(0, K)
        def _(j):
            cv = plsc.load_gather(coef, [jnp.full((L,), j, jnp.int32)])
            for k2 in range(D // L):
                sl2 = pl.ds(k2 * L, L)
                rowbuf[j, sl2] = rowbuf[j, sl2] * cv

        pltpu.sync_copy(rowbuf, acc_sh.at[didx], add=True)

    plsc.subcore_barrier()

    pltpu.sync_copy(acc_sh.at[pl.ds(row0, RPT)],
                    part_hbm.at[c, pl.ds(row0, RPT)])


_sc_gcn = functools.partial(
    pl.kernel,
    out_type=jax.ShapeDtypeStruct((NC, N, D), jnp.float32),
    mesh=plsc.VectorSubcoreMesh(core_axis_name="c", subcore_axis_name="s"),
    scratch_types=[
        pltpu.VMEM_SHARED((N,), jnp.float32),
        pltpu.VMEM_SHARED((N, D), jnp.float32),
        pltpu.VMEM((125, D), jnp.float32),
        pltpu.VMEM((2000,), jnp.float32),
        pltpu.VMEM((K,), jnp.int32),
        pltpu.VMEM((K,), jnp.int32),
        pltpu.VMEM((K,), jnp.float32),
        pltpu.VMEM((K,), jnp.float32),
        pltpu.VMEM((K, D), jnp.float32),
        pltpu.VMEM((N,), jnp.float32),
        pltpu.VMEM((N,), jnp.float32),
    ],
)(_sc_body)


def _mm_body(x_ref, w_ref, o_ref):
    o_ref[...] = jnp.dot(x_ref[...], w_ref[...],
                         preferred_element_type=jnp.float32)


def _comb_body(p_ref, b_ref, o_ref):
    o_ref[...] = p_ref[0] + p_ref[1] + b_ref[...]


def kernel(x, edge_index, edge_weight, W, b):
    src = edge_index[0].astype(jnp.int32)
    dst = edge_index[1].astype(jnp.int32)
    h = pl.pallas_call(
        _mm_body,
        grid=(10,),
        in_specs=[pl.BlockSpec((N // 10, D), lambda i: (i, 0)),
                  pl.BlockSpec((D, D), lambda i: (0, 0))],
        out_specs=pl.BlockSpec((N // 10, D), lambda i: (i, 0)),
        out_shape=jax.ShapeDtypeStruct((N, D), jnp.float32),
    )(x, W)
    part = _sc_gcn(src, dst, edge_weight.astype(jnp.float32), h)
    out = pl.pallas_call(
        _comb_body,
        grid=(8,),
        in_specs=[pl.BlockSpec((NC, N // 8, D), lambda i: (0, i, 0)),
                  pl.BlockSpec((1, D), lambda i: (0, 0))],
        out_specs=pl.BlockSpec((N // 8, D), lambda i: (i, 0)),
        out_shape=jax.ShapeDtypeStruct((N, D), jnp.float32),
    )(part, b.reshape(1, D))
    return out


# speedup vs baseline: 9.0050x; 9.0050x over previous
import functools

import jax
import jax.numpy as jnp
from jax import lax
from jax.experimental import pallas as pl
from jax.experimental.pallas import tpu as pltpu
from jax.experimental.pallas import tpu_sc as plsc

N = 10000
E = 320000
D = 128
NC, NS, L = 2, 16, 16
K = 80
E_CORE = E // NC
E_TILE = E_CORE // NS
E_TILE_DEG = E // NS
DEG_CHUNKS = E_TILE_DEG // K
MAIN_CHUNKS = E_TILE // K
RPT = N // NS


def _sc_body(src_hbm, dst_hbm, ew_hbm, h_hbm, part_hbm,
             deg_sh, acc_sh, z2d, z1d, sidx, didx, ewv, coef, rowbuf,
             degt, dinv):
    c = lax.axis_index("c")
    s = lax.axis_index("s")
    zero16 = jnp.zeros((L,), jnp.float32)

    @pl.loop(0, 125)
    def _(r):
        for k in range(D // L):
            z2d[r, pl.ds(k * L, L)] = zero16

    @pl.loop(0, 2000 // L)
    def _(i):
        z1d[pl.ds(i * L, L)] = zero16

    row0 = s * RPT
    for k in range(RPT // 125):
        pltpu.sync_copy(z2d, acc_sh.at[pl.ds(row0 + k * 125, 125)])

    @pl.when(s == 0)
    def _():
        for k in range(N // 2000):
            pltpu.sync_copy(z1d, deg_sh.at[pl.ds(k * 2000, 2000)])

    plsc.subcore_barrier()

    dbase = s * E_TILE_DEG

    @pl.loop(0, DEG_CHUNKS)
    def _(i):
        off = dbase + i * K
        pltpu.sync_copy(dst_hbm.at[pl.ds(off, K)], didx)
        pltpu.sync_copy(ew_hbm.at[pl.ds(off, K)], ewv)
        for g in range(K // L):
            sl = pl.ds(g * L, L)
            ewv[sl] = jnp.maximum(ewv[sl], 1e-5)
        pltpu.sync_copy(ewv, deg_sh.at[didx], add=True)

    plsc.subcore_barrier()

    pltpu.sync_copy(deg_sh, degt)

    @pl.loop(0, N // L)
    def _(g):
        sl = pl.ds(g * L, L)
        dv = degt[sl]
        x = jnp.maximum(dv, 1e-12)
        i32 = lax.bitcast_convert_type(x, jnp.int32)
        y = lax.bitcast_convert_type(jnp.int32(0x5F3759DF) - (i32 >> 1),
                                     jnp.float32)
        for _ in range(3):
            y = y * (1.5 - 0.5 * x * y * y)
        dinv[sl] = jnp.where(dv > 0.0, y, 0.0)

    mbase = c * E_CORE + s * E_TILE

    @pl.loop(0, MAIN_CHUNKS)
    def _(i):
        off = mbase + i * K
        pltpu.sync_copy(src_hbm.at[pl.ds(off, K)], sidx)
        pltpu.sync_copy(dst_hbm.at[pl.ds(off, K)], didx)
        pltpu.sync_copy(ew_hbm.at[pl.ds(off, K)], ewv)
        pltpu.sync_copy(h_hbm.at[sidx], rowbuf)
        for g in range(K // L):
            sl = pl.ds(g * L, L)
            e = jnp.maximum(ewv[sl], 1e-5)
            a = plsc.load_gather(dinv, [sidx[sl]])
            t = plsc.load_gather(dinv, [didx[sl]])
            coef[sl] = e * a * t

        @pl.loop(0, K)
        def _(j):
            cv = plsc.load_gather(coef, [jnp.full((L,), j, jnp.int32)])
            for k2 in range(D // L):
                sl2 = pl.ds(k2 * L, L)
                rowbuf[j, sl2] = rowbuf[j, sl2] * cv

        pltpu.sync_copy(rowbuf, acc_sh.at[didx], add=True)

    plsc.subcore_barrier()

    @pl.when(s < NS - 1)
    def _():
        pltpu.sync_copy(acc_sh.at[pl.ds(s * 640, 640)],
                        part_hbm.at[c, pl.ds(s * 640, 640)])

    @pl.when(s == NS - 1)
    def _():
        pltpu.sync_copy(acc_sh.at[pl.ds(9600, 400)],
                        part_hbm.at[c, pl.ds(9600, 400)])


_sc_gcn = functools.partial(
    pl.kernel,
    out_type=jax.ShapeDtypeStruct((NC, N, D), jnp.float32),
    mesh=plsc.VectorSubcoreMesh(core_axis_name="c", subcore_axis_name="s"),
    compiler_params=pltpu.CompilerParams(needs_layout_passes=False),
    scratch_types=[
        pltpu.VMEM_SHARED((N,), jnp.float32),
        pltpu.VMEM_SHARED((N, D), jnp.float32),
        pltpu.VMEM((125, D), jnp.float32),
        pltpu.VMEM((2000,), jnp.float32),
        pltpu.VMEM((K,), jnp.int32),
        pltpu.VMEM((K,), jnp.int32),
        pltpu.VMEM((K,), jnp.float32),
        pltpu.VMEM((K,), jnp.float32),
        pltpu.VMEM((K, D), jnp.float32),
        pltpu.VMEM((N,), jnp.float32),
        pltpu.VMEM((N,), jnp.float32),
    ],
)(_sc_body)


def _mm_body(x_ref, w_ref, o_ref):
    o_ref[...] = jnp.dot(x_ref[...], w_ref[...],
                         preferred_element_type=jnp.float32)


def _comb_body(p_ref, b_ref, o_ref):
    o_ref[...] = p_ref[0] + p_ref[1] + b_ref[...]


def kernel(x, edge_index, edge_weight, W, b):
    src = edge_index[0].astype(jnp.int32)
    dst = edge_index[1].astype(jnp.int32)
    h = pl.pallas_call(
        _mm_body,
        grid=(10,),
        in_specs=[pl.BlockSpec((N // 10, D), lambda i: (i, 0)),
                  pl.BlockSpec((D, D), lambda i: (0, 0))],
        out_specs=pl.BlockSpec((N // 10, D), lambda i: (i, 0)),
        out_shape=jax.ShapeDtypeStruct((N, D), jnp.float32),
    )(x, W)
    part = _sc_gcn(src, dst, edge_weight.astype(jnp.float32), h)
    out = pl.pallas_call(
        _comb_body,
        grid=(10,),
        in_specs=[pl.BlockSpec((NC, N // 10, D), lambda i: (0, i, 0)),
                  pl.BlockSpec((1, D), lambda i: (0, 0))],
        out_specs=pl.BlockSpec((N // 10, D), lambda i: (i, 0)),
        out_shape=jax.ShapeDtypeStruct((N, D), jnp.float32),
    )(part, b.reshape(1, D))
    return out


# speedup vs baseline: 25.3945x; 2.8200x over previous
import functools

import jax
import jax.numpy as jnp
from jax import lax
from jax.experimental import pallas as pl
from jax.experimental.pallas import tpu as pltpu
from jax.experimental.pallas import tpu_sc as plsc

N = 10000
E = 320000
D = 128
NC, NS, L = 2, 16, 16
K = 80
E_TILE = E // (NC * NS)
MAIN_CHUNKS = E_TILE // K
NBUF = 3
DGB = 5
DEG_BLOCKS = MAIN_CHUNKS // DGB
NSTRIPE = 640
LAST_STRIPE = N - NSTRIPE * (NS - 1)


def _newton_rsqrt(d):
    x = jnp.maximum(d, 1e-12)
    i32 = lax.bitcast_convert_type(x, jnp.int32)
    y = lax.bitcast_convert_type(jnp.int32(0x5F3759DF) - (i32 >> 1),
                                 jnp.float32)
    for _ in range(3):
        y = y * (1.5 - 0.5 * x * y * y)
    return jnp.where(d > 0.0, y, 0.0)


def _sc_body(srcm_hbm, dstm_hbm, ewm_hbm, h_hbm, part_hbm,
             deg_sh, acc_sh, dinv_sh,
             ddst, dewv, sidx, didx, coefb, z1d, dinv, rowbufs,
             sem_degdma, sems_deg, sems_i, sems_g, sems_s):
    c = lax.axis_index("c")
    s = lax.axis_index("s")
    t = c * NS + s
    zero16 = jnp.zeros((L,), jnp.float32)

    zrow = rowbufs[0]

    @pl.loop(0, K)
    def _(r):
        for k in range(D // L):
            zrow[r, pl.ds(k * L, L)] = zero16

    @pl.loop(0, 1040 // L)
    def _(i):
        z1d[pl.ds(i * L, L)] = zero16

    @pl.when(s < NS - 1)
    def _():
        for k in range(NSTRIPE // K):
            pltpu.sync_copy(zrow, acc_sh.at[pl.ds(s * NSTRIPE + k * K, K)])

    @pl.when(s == NS - 1)
    def _():
        for k in range(LAST_STRIPE // K):
            pltpu.sync_copy(zrow,
                            acc_sh.at[pl.ds((NS - 1) * NSTRIPE + k * K, K)])

    @pl.when(s == 0)
    def _():
        for k in range(N // 1000):
            pltpu.sync_copy(z1d.at[pl.ds(0, 1000)],
                            deg_sh.at[pl.ds(k * 1000, 1000)])

    plsc.subcore_barrier()

    def deg_issue_dma(w, blk, q):
        pltpu.async_copy(dstm_hbm.at[2 * s + w].at[pl.ds(blk * DGB, DGB)],
                         ddst[q], sem_degdma)
        pltpu.async_copy(ewm_hbm.at[2 * s + w].at[pl.ds(blk * DGB, DGB)],
                         dewv[q], sem_degdma)

    def deg_wait_dma(w, blk, q):
        pltpu.make_async_copy(
            dstm_hbm.at[2 * s + w].at[pl.ds(blk * DGB, DGB)],
            ddst[q], sem_degdma).wait()
        pltpu.make_async_copy(
            ewm_hbm.at[2 * s + w].at[pl.ds(blk * DGB, DGB)],
            dewv[q], sem_degdma).wait()

    def deg_process(q):
        @pl.loop(0, DGB)
        def _(r):
            for g in range(K // L):
                sl = pl.ds(g * L, L)
                dewv[q][r, sl] = jnp.maximum(dewv[q][r, sl], 1e-5)

        for r in range(DGB):
            pltpu.async_copy(dewv[q].at[r], deg_sh.at[ddst[q].at[r]],
                             sems_deg[q], add=True)

    def deg_drain(q):
        for r in range(DGB):
            pltpu.make_async_copy(dewv[q].at[r], deg_sh.at[ddst[q].at[r]],
                                  sems_deg[q]).wait()

    deg_issue_dma(0, 0, 0)
    for gblk in range(2 * DEG_BLOCKS):
        w, blk, q = gblk // DEG_BLOCKS, gblk % DEG_BLOCKS, gblk % 2
        deg_wait_dma(w, blk, q)
        if gblk > 0:
            deg_drain(1 - q)
        if gblk + 1 < 2 * DEG_BLOCKS:
            deg_issue_dma((gblk + 1) // DEG_BLOCKS, (gblk + 1) % DEG_BLOCKS,
                          1 - q)
        deg_process(q)
    deg_drain((2 * DEG_BLOCKS - 1) % 2)

    plsc.subcore_barrier()

    @pl.when(s < NS - 1)
    def _():
        pltpu.sync_copy(deg_sh.at[pl.ds(s * NSTRIPE, NSTRIPE)],
                        z1d.at[pl.ds(0, NSTRIPE)])

        @pl.loop(0, NSTRIPE // L)
        def _(g):
            sl = pl.ds(g * L, L)
            z1d[sl] = _newton_rsqrt(z1d[sl])

        pltpu.sync_copy(z1d.at[pl.ds(0, NSTRIPE)],
                        dinv_sh.at[pl.ds(s * NSTRIPE, NSTRIPE)])

    @pl.when(s == NS - 1)
    def _():
        base = (NS - 1) * NSTRIPE
        pltpu.sync_copy(deg_sh.at[pl.ds(base, LAST_STRIPE)],
                        z1d.at[pl.ds(0, LAST_STRIPE)])

        @pl.loop(0, LAST_STRIPE // L)
        def _(g):
            sl = pl.ds(g * L, L)
            z1d[sl] = _newton_rsqrt(z1d[sl])

        pltpu.sync_copy(z1d.at[pl.ds(0, LAST_STRIPE)],
                        dinv_sh.at[pl.ds(base, LAST_STRIPE)])

    plsc.subcore_barrier()
    pltpu.sync_copy(dinv_sh, dinv)

    def issue_idx(i, p):
        pltpu.async_copy(srcm_hbm.at[t].at[i], sidx[p], sems_i[p])
        pltpu.async_copy(dstm_hbm.at[t].at[i], didx[p], sems_i[p])
        pltpu.async_copy(ewm_hbm.at[t].at[i], coefb[p], sems_i[p])

    def wait_idx(i, p):
        for _ in range(3):
            pltpu.make_async_copy(srcm_hbm.at[t].at[i], sidx[p],
                                  sems_i[p]).wait()

    def issue_gather(p):
        pltpu.async_copy(h_hbm.at[sidx[p]], rowbufs[p], sems_g[p])

    def wait_gather(p):
        pltpu.make_async_copy(h_hbm.at[sidx[p]], rowbufs[p], sems_g[p]).wait()

    def issue_scatter(p):
        pltpu.async_copy(rowbufs[p], acc_sh.at[didx[p]], sems_s[p], add=True)

    def wait_scatter(p):
        pltpu.make_async_copy(rowbufs[p], acc_sh.at[didx[p]],
                              sems_s[p]).wait()

    def coef_scale(p):
        buf = rowbufs[p]
        for g in range(K // L):
            sl = pl.ds(g * L, L)
            e = jnp.maximum(coefb[p][sl], 1e-5)
            a = plsc.load_gather(dinv, [sidx[p][sl]])
            bb = plsc.load_gather(dinv, [didx[p][sl]])
            coefb[p][sl] = e * a * bb

        @pl.loop(0, K)
        def _(j):
            cv = plsc.load_gather(coefb[p], [jnp.full((L,), j, jnp.int32)])
            for k2 in range(D // L):
                sl2 = pl.ds(k2 * L, L)
                buf[j, sl2] = buf[j, sl2] * cv

    def body(i, p, wait_prev_scatter, next1, next2):
        p1, p2 = (p + 1) % NBUF, (p + 2) % NBUF
        wait_gather(p)
        if next1:
            wait_idx(i + 1, p1)
            issue_gather(p1)
        coef_scale(p)
        issue_scatter(p)
        if wait_prev_scatter:
            wait_scatter(p2)
        if next2:
            issue_idx(i + 2, p2)

    issue_idx(0, 0)
    issue_idx(1, 1)
    wait_idx(0, 0)
    issue_gather(0)
    body(0, 0, False, True, True)
    body(1, 1, True, True, True)
    body(2, 2, True, True, True)

    @pl.loop(1, MAIN_CHUNKS // NBUF)
    def _(ii):
        for p in range(NBUF):
            body(ii * NBUF + p, p, True, True, True)

    body(MAIN_CHUNKS - 2, (MAIN_CHUNKS - 2) % NBUF, True, True, False)
    body(MAIN_CHUNKS - 1, (MAIN_CHUNKS - 1) % NBUF, True, False, False)
    wait_scatter((MAIN_CHUNKS - 1) % NBUF)

    plsc.subcore_barrier()

    @pl.when(s < NS - 1)
    def _():
        pltpu.sync_copy(acc_sh.at[pl.ds(s * NSTRIPE, NSTRIPE)],
                        part_hbm.at[c, pl.ds(s * NSTRIPE, NSTRIPE)])

    @pl.when(s == NS - 1)
    def _():
        base = (NS - 1) * NSTRIPE
        pltpu.sync_copy(acc_sh.at[pl.ds(base, LAST_STRIPE)],
                        part_hbm.at[c, pl.ds(base, LAST_STRIPE)])


_sc_gcn = functools.partial(
    pl.kernel,
    out_type=jax.ShapeDtypeStruct((NC, N, D), jnp.float32),
    mesh=plsc.VectorSubcoreMesh(core_axis_name="c", subcore_axis_name="s"),
    compiler_params=pltpu.CompilerParams(needs_layout_passes=False,
                                         use_tc_tiling_on_sc=False),
    scratch_types=[
        pltpu.VMEM_SHARED((N,), jnp.float32),
        pltpu.VMEM_SHARED((N, D), jnp.float32),
        pltpu.VMEM_SHARED((N,), jnp.float32),
        [pltpu.VMEM((DGB, K), jnp.int32)] * 2,
        [pltpu.VMEM((DGB, K), jnp.float32)] * 2,
        [pltpu.VMEM((K,), jnp.int32)] * NBUF,
        [pltpu.VMEM((K,), jnp.int32)] * NBUF,
        [pltpu.VMEM((K,), jnp.float32)] * NBUF,
        pltpu.VMEM((1040,), jnp.float32),
        pltpu.VMEM((N,), jnp.float32),
        [pltpu.VMEM((K, D), jnp.float32)] * NBUF,
        pltpu.SemaphoreType.DMA,
        [pltpu.SemaphoreType.DMA] * 2,
        [pltpu.SemaphoreType.DMA] * NBUF,
        [pltpu.SemaphoreType.DMA] * NBUF,
        [pltpu.SemaphoreType.DMA] * NBUF,
    ],
)(_sc_body)


def _mm_body(x_ref, w_ref, o_ref):
    o_ref[...] = jnp.dot(x_ref[...], w_ref[...],
                         preferred_element_type=jnp.float32)


def _comb_body(p_ref, b_ref, o_ref):
    o_ref[...] = p_ref[0] + p_ref[1] + b_ref[...]


def kernel(x, edge_index, edge_weight, W, b):
    src = edge_index[0].astype(jnp.int32)
    dst = edge_index[1].astype(jnp.int32)
    ew = edge_weight.astype(jnp.float32)
    h = pl.pallas_call(
        _mm_body,
        grid=(10,),
        in_specs=[pl.BlockSpec((N // 10, D), lambda i: (i, 0)),
                  pl.BlockSpec((D, D), lambda i: (0, 0))],
        out_specs=pl.BlockSpec((N // 10, D), lambda i: (i, 0)),
        out_shape=jax.ShapeDtypeStruct((N, D), jnp.float32),
    )(x, W)
    part = _sc_gcn(
        src.reshape(NC * NS, MAIN_CHUNKS, K),
        dst.reshape(NC * NS, MAIN_CHUNKS, K),
        ew.reshape(NC * NS, MAIN_CHUNKS, K),
        h,
    )
    out = pl.pallas_call(
        _comb_body,
        grid=(10,),
        in_specs=[pl.BlockSpec((NC, N // 10, D), lambda i: (0, i, 0)),
                  pl.BlockSpec((1, D), lambda i: (0, 0))],
        out_specs=pl.BlockSpec((N // 10, D), lambda i: (i, 0)),
        out_shape=jax.ShapeDtypeStruct((N, D), jnp.float32),
    )(part, b.reshape(1, D))
    return out
